# Initial kernel scaffold; baseline (speedup 1.0000x reference)
import jax
import jax.numpy as jnp
from jax.experimental import pallas as pl


def kernel(x, emb_weight):
    raise NotImplementedError("write your pallas kernel here")


# speedup vs baseline: 4.5688x; 4.5688x over previous
import jax
import jax.numpy as jnp
from jax.experimental import pallas as pl
from jax.experimental.pallas import tpu as pltpu

B, F, N, K = 8, 64, 256, 512


def _vq_body(x_ref, emb_ref, q_ref, dist_ref):
    xb = x_ref[0]
    emb = emb_ref[...]
    xtb = xb.T
    embt = emb.T
    dot = jax.lax.dot_general(xtb, embt, (((1,), (0,)), ((), ())),
                              precision=jax.lax.Precision.HIGHEST,
                              preferred_element_type=jnp.float32)
    c2 = jnp.sum(embt * embt, axis=0, keepdims=True)
    x2 = jnp.sum(xtb * xtb, axis=1, keepdims=True)
    g = c2 - 2.0 * dot
    dist_ref[0] = g + x2
    minv = jnp.min(g, axis=1, keepdims=True)
    iota = jax.lax.broadcasted_iota(jnp.int32, (N, K), 1)
    idx = jnp.min(jnp.where(g == minv, iota, K), axis=1, keepdims=True)
    oh = (iota == idx).astype(jnp.float32)
    q_nf = jax.lax.dot_general(oh, emb, (((1,), (0,)), ((), ())),
                               precision=jax.lax.Precision.HIGHEST,
                               preferred_element_type=jnp.float32)
    q_ref[0] = q_nf.T


def kernel(x, emb_weight):
    q, dist = pl.pallas_call(
        _vq_body,
        grid=(B,),
        in_specs=[
            pl.BlockSpec((1, F, N), lambda b: (b, 0, 0)),
            pl.BlockSpec((K, F), lambda b: (0, 0)),
        ],
        out_specs=[
            pl.BlockSpec((1, F, N), lambda b: (b, 0, 0)),
            pl.BlockSpec((1, N, K), lambda b: (b, 0, 0)),
        ],
        out_shape=[
            jax.ShapeDtypeStruct((B, F, N), jnp.float32),
            jax.ShapeDtypeStruct((B, N, K), jnp.float32),
        ],
    )(x, emb_weight)
    return q, dist
